# Initial kernel scaffold; baseline (speedup 1.0000x reference)
import jax
import jax.numpy as jnp
from jax.experimental import pallas as pl


def kernel(lhs, rhs, m_indices):
    raise NotImplementedError("write your pallas kernel here")


# speedup vs baseline: 1.9184x; 1.9184x over previous
import jax
import jax.numpy as jnp
from jax.experimental import pallas as pl
from jax.experimental.pallas import tpu as pltpu

_G = 64
_N = 1024
_K = 4096
_M = 4096
_TM = 128
_NUM_TILES = 96
_HALF = _NUM_TILES // 2


def _gmm_body(tile_group_ref, num_tiles_ref, x_ref, w_ref, o_ref):
    del tile_group_ref
    s = pl.program_id(0) * _HALF + pl.program_id(1)

    @pl.when(s < num_tiles_ref[0])
    def _():
        acc = jax.lax.dot_general(
            x_ref[...], w_ref[0],
            (((1,), (1,)), ((), ())),
            preferred_element_type=jnp.float32)
        o_ref[...] = acc.astype(jnp.bfloat16)


def _grouped_matmul(tile_group, num_tiles, lhs_slots, rhs):
    grid_spec = pltpu.PrefetchScalarGridSpec(
        num_scalar_prefetch=2,
        grid=(2, _HALF),
        in_specs=[
            pl.BlockSpec((_TM, _K), lambda c, i, tg, nt: (c * _HALF + i, 0)),
            pl.BlockSpec((1, _N, _K),
                         lambda c, i, tg, nt: (tg[c * _HALF + i], 0, 0)),
        ],
        out_specs=pl.BlockSpec((_TM, _N), lambda c, i, tg, nt: (c * _HALF + i, 0)),
    )
    return pl.pallas_call(
        _gmm_body,
        out_shape=jax.ShapeDtypeStruct((_NUM_TILES * _TM, _N), jnp.bfloat16),
        grid_spec=grid_spec,
        compiler_params=pltpu.CompilerParams(
            dimension_semantics=("parallel", "arbitrary")),
        name="grouped_matmul",
    )(tile_group, num_tiles, lhs_slots, rhs)


def kernel(lhs, rhs, m_indices):
    m_indices = m_indices.astype(jnp.int32)

    counts = jnp.bincount(m_indices, length=_G).astype(jnp.int32)
    sort_idx = jnp.argsort(m_indices).astype(jnp.int32)
    row_start = (jnp.cumsum(counts) - counts).astype(jnp.int32)

    tiles_pg = (counts + _TM - 1) // _TM
    tile_cum = jnp.cumsum(tiles_pg).astype(jnp.int32)
    tile_start = (tile_cum - tiles_pg).astype(jnp.int32)
    num_tiles = tile_cum[_G - 1]

    s_ar = jnp.arange(_NUM_TILES, dtype=jnp.int32)
    raw_g = jnp.clip(
        jnp.searchsorted(tile_cum, s_ar, side='right'), 0, _G - 1
    ).astype(jnp.int32)
    last_g = raw_g[jnp.maximum(num_tiles - 1, 0)]
    tile_group = jnp.where(s_ar < num_tiles, raw_g, last_g).astype(jnp.int32)

    g = tile_group
    local = (s_ar - tile_start[g])[:, None] * _TM + jnp.arange(
        _TM, dtype=jnp.int32)[None, :]
    local = jnp.minimum(local, jnp.maximum(counts[g] - 1, 0)[:, None])
    row_ids = sort_idx[jnp.minimum(row_start[g][:, None] + local, _M - 1)]

    ranks = jnp.arange(_M, dtype=jnp.int32)
    g_of_rank = m_indices[sort_idx]
    slot_sorted = tile_start[g_of_rank] * _TM + (ranks - row_start[g_of_rank])
    slot_of_row = jnp.zeros((_M,), jnp.int32).at[sort_idx].set(slot_sorted)

    lhs_slots = jnp.take(lhs, row_ids.reshape(-1), axis=0)
    out_slots = _grouped_matmul(tile_group, num_tiles.reshape(1),
                                lhs_slots, rhs)
    return jnp.take(out_slots, slot_of_row, axis=0)
